# baseline (speedup 1.0000x reference)
import functools

import jax
import jax.numpy as jnp
from jax import lax
from jax.experimental import pallas as pl
from jax.experimental.pallas import tpu as pltpu
from jax.experimental.pallas import tpu_sc as plsc

N_I = 256
N_P = 8192
N_C = 9
I_BLK = 32
NW = 32
I_W = N_I // NW
CP = 4096
NH = N_P // CP
LANES = 16


def _rot_plane_kernel(w_ref, x_ref, o_ref):
    w = w_ref[...]
    for d in range(3):
        acc = x_ref[0] * w[:, 3 * d : 3 * d + 1]
        acc += x_ref[1] * w[:, 3 * d + 1 : 3 * d + 2]
        acc += x_ref[2] * w[:, 3 * d + 2 : 3 * d + 3]
        o_ref[d] = acc


def _sc_copy_body(x_hbm, o_hbm, cpb, clsem, cssem):
    wid = lax.axis_index("s") * 2 + lax.axis_index("c")
    base = wid * I_W

    def cp_load(t, slot):
        plane = 3 + t // NH
        half = lax.rem(t, NH)
        return pltpu.make_async_copy(
            x_hbm.at[plane, pl.ds(base, I_W), pl.ds(half * CP, CP)],
            cpb.at[slot], clsem.at[slot])

    def cp_store(t, slot):
        plane = t // NH
        half = lax.rem(t, NH)
        return pltpu.make_async_copy(
            cpb.at[slot],
            o_hbm.at[plane, pl.ds(base, I_W), pl.ds(half * CP, CP)],
            cssem.at[slot])

    def cp_t(t, carry):
        slot = lax.rem(t, 2)
        @pl.when(t >= 2)
        def _():
            cp_store(t - 2, slot).wait()
        cp_load(t, slot).start()
        cp_load(t, slot).wait()
        cp_store(t, slot).start()
        return carry

    n_t = (N_C - 3) * NH
    lax.fori_loop(0, n_t, cp_t, 0)
    cp_store(n_t - 2, lax.rem(n_t - 2, 2)).wait()
    cp_store(n_t - 1, lax.rem(n_t - 1, 2)).wait()


@functools.partial(jax.jit, static_argnames=("interpret",))
def kernel(points_colored_instance, rot_mats, interpret=False):
    xt = jnp.transpose(points_colored_instance, (2, 0, 1))
    w = rot_mats.reshape(N_I, 9)
    rot = pl.pallas_call(
        _rot_plane_kernel,
        grid=(N_I // I_BLK,),
        in_specs=[
            pl.BlockSpec((I_BLK, 9), lambda i: (i, 0)),
            pl.BlockSpec((3, I_BLK, N_P), lambda i: (0, i, 0)),
        ],
        out_specs=pl.BlockSpec((3, I_BLK, N_P), lambda i: (0, i, 0)),
        out_shape=jax.ShapeDtypeStruct((3, N_I, N_P), jnp.float32),
        interpret=interpret,
    )(w, xt)
    mesh = plsc.VectorSubcoreMesh(core_axis_name="c", subcore_axis_name="s")
    cp = pl.kernel(
        _sc_copy_body,
        out_type=jax.ShapeDtypeStruct((N_C - 3, N_I, N_P), jnp.float32),
        mesh=mesh,
        scratch_types=[
            pltpu.VMEM((2, I_W, CP), jnp.float32),
            pltpu.SemaphoreType.DMA((2,)),
            pltpu.SemaphoreType.DMA((2,)),
        ],
        interpret=interpret,
    )(xt)
    out = jnp.concatenate([rot, cp], axis=0)
    return jnp.transpose(out, (1, 2, 0))


# speedup vs baseline: 2.3209x; 2.3209x over previous
import functools

import jax
import jax.numpy as jnp
from jax.experimental import pallas as pl

N_I = 256
N_P = 8192
N_C = 9
I_BLK = 32
P_BLK = 8192


def _rot_plane_kernel(w_ref, x_ref, o_ref):
    w = w_ref[...]
    for d in range(3):
        acc = x_ref[0] * w[:, 3 * d : 3 * d + 1]
        acc += x_ref[1] * w[:, 3 * d + 1 : 3 * d + 2]
        acc += x_ref[2] * w[:, 3 * d + 2 : 3 * d + 3]
        o_ref[d] = acc
    for c in range(3, N_C):
        o_ref[c] = x_ref[c]


@functools.partial(jax.jit, static_argnames=("interpret",))
def kernel(points_colored_instance, rot_mats, interpret=False):
    xt = jnp.transpose(points_colored_instance, (2, 0, 1))
    w = rot_mats.reshape(N_I, 9)
    out = pl.pallas_call(
        _rot_plane_kernel,
        grid=(N_I // I_BLK, N_P // P_BLK),
        in_specs=[
            pl.BlockSpec((I_BLK, 9), lambda i, j: (i, 0)),
            pl.BlockSpec((N_C, I_BLK, P_BLK), lambda i, j: (0, i, j)),
        ],
        out_specs=pl.BlockSpec((N_C, I_BLK, P_BLK), lambda i, j: (0, i, j)),
        out_shape=jax.ShapeDtypeStruct((N_C, N_I, N_P), jnp.float32),
        interpret=interpret,
    )(w, xt)
    return jnp.transpose(out, (1, 2, 0))
